# Initial kernel scaffold; baseline (speedup 1.0000x reference)
import jax
import jax.numpy as jnp
from jax.experimental import pallas as pl


def kernel(h, edge_index, etypes, W1, Ws1, b1, gamma1, beta1, rm1, rv1, W2, Ws2, b2, gamma2, beta2, rm2, rv2):
    raise NotImplementedError("write your pallas kernel here")


# speedup vs baseline: 20.2135x; 20.2135x over previous
import jax
import jax.numpy as jnp
from jax import lax
from jax.experimental import pallas as pl
from jax.experimental.pallas import tpu as pltpu
from jax.experimental.pallas import tpu_sc as plsc

N = 10000
NP = 10240
E = 320000
D = 128
R = 5
RT = R + 1

NC = 2
NS = 16
NW = NC * NS
EPW = E // NW
CH = 80
SCH = 2000
NCH_S = SCH // CH
NSUP = EPW // SCH
RPT = NP // NS
ZR = 80

BN_BLK = 1024
NB = NP // BN_BLK


def _transform_body(x_ref, w_ref, o_ref):
    o_ref[0] = jnp.dot(x_ref[...], w_ref[0], preferred_element_type=jnp.float32)


@jax.jit
def _transform(x, wall):
    return pl.pallas_call(
        _transform_body,
        grid=(NB, RT),
        in_specs=[
            pl.BlockSpec((BN_BLK, D), lambda i, r: (i, 0)),
            pl.BlockSpec((1, D, D), lambda i, r: (r, 0, 0)),
        ],
        out_specs=pl.BlockSpec((1, BN_BLK, D), lambda i, r: (r, i, 0)),
        out_shape=jax.ShapeDtypeStruct((RT, NP, D), jnp.float32),
    )(x, wall)


def _combine_body(a0, a1, sl, b, g, be, rm, rv, o):
    z = a0[...] + a1[...] + sl[...] + b[...]
    z = jnp.where(z > 0, z, 0.01 * z)
    inv = lax.rsqrt(rv[...] + 1e-5)
    y = (z - rm[...]) * inv * g[...] + be[...]
    o[...] = jnp.maximum(y, 0.0)


@jax.jit
def _combine(a0, a1, sl, b, g, be, rm, rv):
    row = pl.BlockSpec((BN_BLK, D), lambda i: (i, 0))
    par = pl.BlockSpec((1, D), lambda i: (0, 0))
    return pl.pallas_call(
        _combine_body,
        grid=(NB,),
        in_specs=[row, row, row, par, par, par, par, par],
        out_specs=row,
        out_shape=jax.ShapeDtypeStruct((NP, D), jnp.float32),
    )(a0, a1, sl, b.reshape(1, D), g.reshape(1, D), be.reshape(1, D),
      rm.reshape(1, D), rv.reshape(1, D))


def _edge_body(hw, srch, dst4h, eth, out, src_v, et_v, gidx_v, didx_v,
               rows_v, agg_sh):
    c = lax.axis_index("c")
    s = lax.axis_index("s")
    wid = s * NC + c

    zvec = jnp.zeros((16,), jnp.float32)

    @pl.loop(0, ZR)
    def _(i):
        @pl.loop(0, D, step=16)
        def _(j):
            rows_v[i, pl.ds(j, 16)] = zvec

    base_row = s * RPT

    @pl.loop(0, RPT, step=ZR)
    def _(r):
        pltpu.sync_copy(rows_v, agg_sh.at[pl.ds(base_row + r, ZR)])

    plsc.subcore_barrier()

    ebase = wid * EPW

    @pl.loop(0, NSUP)
    def _(sc):
        pltpu.sync_copy(srch.at[pl.ds(ebase + sc * SCH, SCH)], src_v)
        pltpu.sync_copy(eth.at[pl.ds(ebase + sc * SCH, SCH)], et_v)
        pltpu.sync_copy(dst4h.at[wid, sc], didx_v)

        @pl.loop(0, NCH_S)
        def _(i):
            @pl.loop(0, CH, step=16)
            def _(j):
                e = i * CH + j
                gidx_v[i, pl.ds(j, 16)] = (et_v[pl.ds(e, 16)] * NP
                                           + src_v[pl.ds(e, 16)])

        @pl.loop(0, NCH_S)
        def _(i):
            pltpu.sync_copy(hw.at[gidx_v.at[i]], rows_v)
            pltpu.sync_copy(rows_v, agg_sh.at[didx_v.at[i]], add=True)

    plsc.subcore_barrier()

    pltpu.sync_copy(agg_sh.at[pl.ds(base_row, RPT)],
                    out.at[c, pl.ds(base_row, RPT)])


@jax.jit
def _edge_pass(hw_flat, src, dst4, etypes):
    mesh = plsc.VectorSubcoreMesh(core_axis_name="c", subcore_axis_name="s",
                                  num_cores=NC, num_subcores=NS)
    f = pl.kernel(
        _edge_body,
        out_type=jax.ShapeDtypeStruct((NC, NP, D), jnp.float32),
        mesh=mesh,
        scratch_types=[
            pltpu.VMEM((SCH,), jnp.int32),
            pltpu.VMEM((SCH,), jnp.int32),
            pltpu.VMEM((NCH_S, CH), jnp.int32),
            pltpu.VMEM((NCH_S, CH), jnp.int32),
            pltpu.VMEM((ZR, D), jnp.float32),
            pltpu.VMEM_SHARED((NP, D), jnp.float32),
        ],
    )
    return f(hw_flat, src, dst4, etypes)


def kernel(h, edge_index, etypes, W1, Ws1, b1, gamma1, beta1, rm1, rv1,
           W2, Ws2, b2, gamma2, beta2, rm2, rv2):
    src = edge_index[0]
    dst4 = edge_index[1].reshape(NW, NSUP, NCH_S, CH)

    wall1 = jnp.concatenate([W1, Ws1[None]], axis=0)
    wall2 = jnp.concatenate([W2, Ws2[None]], axis=0)

    hp = jnp.pad(h, ((0, NP - N), (0, 0)))

    hw1 = _transform(hp, wall1)
    agg1 = _edge_pass(hw1.reshape(RT * NP, D), src, dst4, etypes)
    x = _combine(agg1[0], agg1[1], hw1[R], b1, gamma1, beta1, rm1, rv1)

    hw2 = _transform(x, wall2)
    agg2 = _edge_pass(hw2.reshape(RT * NP, D), src, dst4, etypes)
    out = _combine(agg2[0], agg2[1], hw2[R], b2, gamma2, beta2, rm2, rv2)
    return out[:N]


# speedup vs baseline: 28.1377x; 1.3920x over previous
import jax
import jax.numpy as jnp
from jax import lax
from jax.experimental import pallas as pl
from jax.experimental.pallas import tpu as pltpu
from jax.experimental.pallas import tpu_sc as plsc

N = 10000
NP = 10240
E = 320000
D = 128
R = 5
RT = R + 1

NC = 2
NS = 16
NW = NC * NS
EPW = E // NW
CH = 80
SCH = 2000
NCH_S = SCH // CH
NSUP = EPW // SCH
RPT = NP // NS
ZR = 80

BN_BLK = 1024
NB = NP // BN_BLK


def _transform_body(x_ref, w_ref, o_ref):
    o_ref[0] = jnp.dot(x_ref[...], w_ref[0], preferred_element_type=jnp.float32)


@jax.jit
def _transform(x, wall):
    return pl.pallas_call(
        _transform_body,
        grid=(NB, RT),
        in_specs=[
            pl.BlockSpec((BN_BLK, D), lambda i, r: (i, 0)),
            pl.BlockSpec((1, D, D), lambda i, r: (r, 0, 0)),
        ],
        out_specs=pl.BlockSpec((1, BN_BLK, D), lambda i, r: (r, i, 0)),
        out_shape=jax.ShapeDtypeStruct((RT, NP, D), jnp.float32),
    )(x, wall)


def _combine_body(a0, a1, sl, b, g, be, rm, rv, o):
    z = a0[...] + a1[...] + sl[...] + b[...]
    z = jnp.where(z > 0, z, 0.01 * z)
    inv = lax.rsqrt(rv[...] + 1e-5)
    y = (z - rm[...]) * inv * g[...] + be[...]
    o[...] = jnp.maximum(y, 0.0)


@jax.jit
def _combine(a0, a1, sl, b, g, be, rm, rv):
    row = pl.BlockSpec((BN_BLK, D), lambda i: (i, 0))
    par = pl.BlockSpec((1, D), lambda i: (0, 0))
    return pl.pallas_call(
        _combine_body,
        grid=(NB,),
        in_specs=[row, row, row, par, par, par, par, par],
        out_specs=row,
        out_shape=jax.ShapeDtypeStruct((NP, D), jnp.float32),
    )(a0, a1, sl, b.reshape(1, D), g.reshape(1, D), be.reshape(1, D),
      rm.reshape(1, D), rv.reshape(1, D))


def _edge_body(hw, srch, dst4h, eth, out, src_v, et_v, gidx_v, didx_v,
               rows_a, rows_b, sem_a, sem_b, agg_sh):
    c = lax.axis_index("c")
    s = lax.axis_index("s")
    wid = s * NC + c

    zvec = jnp.zeros((16,), jnp.float32)

    @pl.loop(0, ZR)
    def _(i):
        @pl.loop(0, D, step=16)
        def _(j):
            rows_a[i, pl.ds(j, 16)] = zvec

    base_row = s * RPT

    @pl.loop(0, RPT, step=ZR)
    def _(r):
        pltpu.sync_copy(rows_a.at[pl.ds(0, ZR)],
                        agg_sh.at[pl.ds(base_row + r, ZR)])

    plsc.subcore_barrier()

    ebase = wid * EPW

    @pl.loop(0, NSUP)
    def _(sc):
        pltpu.sync_copy(srch.at[pl.ds(ebase + sc * SCH, SCH)], src_v)
        pltpu.sync_copy(eth.at[pl.ds(ebase + sc * SCH, SCH)], et_v)
        pltpu.sync_copy(dst4h.at[wid, sc], didx_v)

        @pl.loop(0, SCH, step=16)
        def _(j):
            gidx_v[pl.ds(j, 16)] = et_v[pl.ds(j, 16)] * NP + src_v[pl.ds(j, 16)]

        def gather(i, buf, sem):
            pltpu.async_copy(hw.at[gidx_v.at[pl.ds(i * CH, CH)]], buf, sem)

        def wait(buf, sem):
            pltpu.make_async_copy(hw.at[pl.ds(0, CH)], buf, sem).wait()

        def scatter(i, buf):
            pltpu.sync_copy(buf, agg_sh.at[didx_v.at[i]], add=True)

        gather(0, rows_a, sem_a)

        @pl.loop(0, NCH_S - 1, step=2)
        def _(i):
            gather(i + 1, rows_b, sem_b)
            wait(rows_a, sem_a)
            scatter(i, rows_a)
            gather(i + 2, rows_a, sem_a)
            wait(rows_b, sem_b)
            scatter(i + 1, rows_b)

        wait(rows_a, sem_a)
        scatter(NCH_S - 1, rows_a)

    plsc.subcore_barrier()

    pltpu.sync_copy(agg_sh.at[pl.ds(base_row, RPT)],
                    out.at[c, pl.ds(base_row, RPT)])


@jax.jit
def _edge_pass(hw_flat, src, dst4, etypes):
    mesh = plsc.VectorSubcoreMesh(core_axis_name="c", subcore_axis_name="s",
                                  num_cores=NC, num_subcores=NS)
    f = pl.kernel(
        _edge_body,
        out_type=jax.ShapeDtypeStruct((NC, NP, D), jnp.float32),
        mesh=mesh,
        scratch_types=[
            pltpu.VMEM((SCH,), jnp.int32),
            pltpu.VMEM((SCH,), jnp.int32),
            pltpu.VMEM((SCH,), jnp.int32),
            pltpu.VMEM((NCH_S, CH), jnp.int32),
            pltpu.VMEM((CH, D), jnp.float32),
            pltpu.VMEM((CH, D), jnp.float32),
            pltpu.SemaphoreType.DMA,
            pltpu.SemaphoreType.DMA,
            pltpu.VMEM_SHARED((NP, D), jnp.float32),
        ],
    )
    return f(hw_flat, src, dst4, etypes)


def kernel(h, edge_index, etypes, W1, Ws1, b1, gamma1, beta1, rm1, rv1,
           W2, Ws2, b2, gamma2, beta2, rm2, rv2):
    src = edge_index[0]
    dst4 = edge_index[1].reshape(NW, NSUP, NCH_S, CH)

    wall1 = jnp.concatenate([W1, Ws1[None]], axis=0)
    wall2 = jnp.concatenate([W2, Ws2[None]], axis=0)

    hp = jnp.pad(h, ((0, NP - N), (0, 0)))

    hw1 = _transform(hp, wall1)
    agg1 = _edge_pass(hw1.reshape(RT * NP, D), src, dst4, etypes)
    x = _combine(agg1[0], agg1[1], hw1[R], b1, gamma1, beta1, rm1, rv1)

    hw2 = _transform(x, wall2)
    agg2 = _edge_pass(hw2.reshape(RT * NP, D), src, dst4, etypes)
    out = _combine(agg2[0], agg2[1], hw2[R], b2, gamma2, beta2, rm2, rv2)
    return out[:N]


# speedup vs baseline: 31.0875x; 1.1048x over previous
import jax
import jax.numpy as jnp
from jax import lax
from jax.experimental import pallas as pl
from jax.experimental.pallas import tpu as pltpu
from jax.experimental.pallas import tpu_sc as plsc

N = 10000
NP = 10240
E = 320000
D = 128
R = 5
RT = R + 1

NC = 2
NS = 16
NW = NC * NS
EPW = E // NW
CH = 80
SCH = 2000
NCH_S = SCH // CH
NSUP = EPW // SCH
RPT = NP // NS
ZR = 80

BN_BLK = 1024
NB = NP // BN_BLK


def _transform_body(x_ref, w_ref, o_ref):
    o_ref[0] = jnp.dot(x_ref[...], w_ref[0], preferred_element_type=jnp.float32)


@jax.jit
def _transform(x, wall):
    return pl.pallas_call(
        _transform_body,
        grid=(NB, RT),
        in_specs=[
            pl.BlockSpec((BN_BLK, D), lambda i, r: (i, 0)),
            pl.BlockSpec((1, D, D), lambda i, r: (r, 0, 0)),
        ],
        out_specs=pl.BlockSpec((1, BN_BLK, D), lambda i, r: (r, i, 0)),
        out_shape=jax.ShapeDtypeStruct((RT, NP, D), jnp.float32),
    )(x, wall)


def _combine_body(a0, a1, sl, b, g, be, rm, rv, o):
    z = a0[...] + a1[...] + sl[...] + b[...]
    z = jnp.where(z > 0, z, 0.01 * z)
    inv = lax.rsqrt(rv[...] + 1e-5)
    y = (z - rm[...]) * inv * g[...] + be[...]
    o[...] = jnp.maximum(y, 0.0)


@jax.jit
def _combine(a0, a1, sl, b, g, be, rm, rv):
    row = pl.BlockSpec((BN_BLK, D), lambda i: (i, 0))
    par = pl.BlockSpec((1, D), lambda i: (0, 0))
    return pl.pallas_call(
        _combine_body,
        grid=(NB,),
        in_specs=[row, row, row, par, par, par, par, par],
        out_specs=row,
        out_shape=jax.ShapeDtypeStruct((NP, D), jnp.float32),
    )(a0, a1, sl, b.reshape(1, D), g.reshape(1, D), be.reshape(1, D),
      rm.reshape(1, D), rv.reshape(1, D))


def _edge_body(hw, srch, dst4h, eth, out, src_v, et_v, gidx_v, didx_v,
               rows_a, rows_b, rows_c, sem_a, sem_b, sem_c, agg_sh):
    c = lax.axis_index("c")
    s = lax.axis_index("s")
    wid = s * NC + c

    zvec = jnp.zeros((16,), jnp.float32)

    @pl.loop(0, ZR)
    def _(i):
        @pl.loop(0, D, step=16)
        def _(j):
            rows_a[i, pl.ds(j, 16)] = zvec

    base_row = s * RPT

    @pl.loop(0, RPT, step=ZR)
    def _(r):
        pltpu.sync_copy(rows_a.at[pl.ds(0, ZR)],
                        agg_sh.at[pl.ds(base_row + r, ZR)])

    plsc.subcore_barrier()

    ebase = wid * EPW

    @pl.loop(0, NSUP)
    def _(sc):
        pltpu.sync_copy(srch.at[pl.ds(ebase + sc * SCH, SCH)], src_v)
        pltpu.sync_copy(eth.at[pl.ds(ebase + sc * SCH, SCH)], et_v)
        pltpu.sync_copy(dst4h.at[wid, sc], didx_v)

        @pl.loop(0, SCH, step=16)
        def _(j):
            gidx_v[pl.ds(j, 16)] = et_v[pl.ds(j, 16)] * NP + src_v[pl.ds(j, 16)]

        def gather(i, buf, sem):
            pltpu.async_copy(hw.at[gidx_v.at[pl.ds(i * CH, CH)]], buf, sem)

        def wait(buf, sem):
            pltpu.make_async_copy(hw.at[pl.ds(0, CH)], buf, sem).wait()

        def scatter(i, buf):
            pltpu.sync_copy(buf, agg_sh.at[didx_v.at[i]], add=True)

        gather(0, rows_a, sem_a)
        gather(1, rows_b, sem_b)

        @pl.loop(0, NCH_S - 6, step=3)
        def _(i):
            gather(i + 2, rows_c, sem_c)
            wait(rows_a, sem_a)
            scatter(i, rows_a)
            gather(i + 3, rows_a, sem_a)
            wait(rows_b, sem_b)
            scatter(i + 1, rows_b)
            gather(i + 4, rows_b, sem_b)
            wait(rows_c, sem_c)
            scatter(i + 2, rows_c)

        gather(NCH_S - 2, rows_c, sem_c)
        wait(rows_a, sem_a)
        scatter(NCH_S - 4, rows_a)
        gather(NCH_S - 1, rows_a, sem_a)
        wait(rows_b, sem_b)
        scatter(NCH_S - 3, rows_b)
        wait(rows_c, sem_c)
        scatter(NCH_S - 2, rows_c)
        wait(rows_a, sem_a)
        scatter(NCH_S - 1, rows_a)

    plsc.subcore_barrier()

    pltpu.sync_copy(agg_sh.at[pl.ds(base_row, RPT)],
                    out.at[c, pl.ds(base_row, RPT)])


@jax.jit
def _edge_pass(hw_flat, src, dst4, etypes):
    mesh = plsc.VectorSubcoreMesh(core_axis_name="c", subcore_axis_name="s",
                                  num_cores=NC, num_subcores=NS)
    f = pl.kernel(
        _edge_body,
        out_type=jax.ShapeDtypeStruct((NC, NP, D), jnp.float32),
        mesh=mesh,
        scratch_types=[
            pltpu.VMEM((SCH,), jnp.int32),
            pltpu.VMEM((SCH,), jnp.int32),
            pltpu.VMEM((SCH,), jnp.int32),
            pltpu.VMEM((NCH_S, CH), jnp.int32),
            pltpu.VMEM((CH, D), jnp.float32),
            pltpu.VMEM((CH, D), jnp.float32),
            pltpu.VMEM((CH, D), jnp.float32),
            pltpu.SemaphoreType.DMA,
            pltpu.SemaphoreType.DMA,
            pltpu.SemaphoreType.DMA,
            pltpu.VMEM_SHARED((NP, D), jnp.float32),
        ],
    )
    return f(hw_flat, src, dst4, etypes)


def kernel(h, edge_index, etypes, W1, Ws1, b1, gamma1, beta1, rm1, rv1,
           W2, Ws2, b2, gamma2, beta2, rm2, rv2):
    src = edge_index[0]
    dst4 = edge_index[1].reshape(NW, NSUP, NCH_S, CH)

    wall1 = jnp.concatenate([W1, Ws1[None]], axis=0)
    wall2 = jnp.concatenate([W2, Ws2[None]], axis=0)

    hp = jnp.pad(h, ((0, NP - N), (0, 0)))

    hw1 = _transform(hp, wall1)
    agg1 = _edge_pass(hw1.reshape(RT * NP, D), src, dst4, etypes)
    x = _combine(agg1[0], agg1[1], hw1[R], b1, gamma1, beta1, rm1, rv1)

    hw2 = _transform(x, wall2)
    agg2 = _edge_pass(hw2.reshape(RT * NP, D), src, dst4, etypes)
    out = _combine(agg2[0], agg2[1], hw2[R], b2, gamma2, beta2, rm2, rv2)
    return out[:N]


# speedup vs baseline: 31.4261x; 1.0109x over previous
import jax
import jax.numpy as jnp
from jax import lax
from jax.experimental import pallas as pl
from jax.experimental.pallas import tpu as pltpu
from jax.experimental.pallas import tpu_sc as plsc

N = 10000
NP = 10240
E = 320000
D = 128
R = 5
RT = R + 1

NC = 2
NS = 16
NW = NC * NS
EPW = E // NW
CH = 80
SCH = 2000
NCH_S = SCH // CH
NSUP = EPW // SCH
RPT = NP // NS
ZR = 80

BN_BLK = 1024
NB = NP // BN_BLK


def _transform_body(x_ref, w_ref, o_ref):
    o_ref[0] = jnp.dot(x_ref[...], w_ref[0], preferred_element_type=jnp.float32)


@jax.jit
def _transform(x, wall):
    return pl.pallas_call(
        _transform_body,
        grid=(NB, RT),
        in_specs=[
            pl.BlockSpec((BN_BLK, D), lambda i, r: (i, 0)),
            pl.BlockSpec((1, D, D), lambda i, r: (r, 0, 0)),
        ],
        out_specs=pl.BlockSpec((1, BN_BLK, D), lambda i, r: (r, i, 0)),
        out_shape=jax.ShapeDtypeStruct((RT, NP, D), jnp.float32),
    )(x, wall)


def _combine_body(a0, a1, sl, b, g, be, rm, rv, o):
    z = a0[...] + a1[...] + sl[...] + b[...]
    z = jnp.where(z > 0, z, 0.01 * z)
    inv = lax.rsqrt(rv[...] + 1e-5)
    y = (z - rm[...]) * inv * g[...] + be[...]
    o[...] = jnp.maximum(y, 0.0)


@jax.jit
def _combine(a0, a1, sl, b, g, be, rm, rv):
    row = pl.BlockSpec((BN_BLK, D), lambda i: (i, 0))
    par = pl.BlockSpec((1, D), lambda i: (0, 0))
    return pl.pallas_call(
        _combine_body,
        grid=(NB,),
        in_specs=[row, row, row, par, par, par, par, par],
        out_specs=row,
        out_shape=jax.ShapeDtypeStruct((NP, D), jnp.float32),
    )(a0, a1, sl, b.reshape(1, D), g.reshape(1, D), be.reshape(1, D),
      rm.reshape(1, D), rv.reshape(1, D))


def _comb_transform_body(a0, a1, sl, b, g, be, rm, rv, w_ref, o_ref, x_blk):
    r = pl.program_id(1)

    @pl.when(r == 0)
    def _():
        z = a0[...] + a1[...] + sl[...] + b[...]
        z = jnp.where(z > 0, z, 0.01 * z)
        inv = lax.rsqrt(rv[...] + 1e-5)
        y = (z - rm[...]) * inv * g[...] + be[...]
        x_blk[...] = jnp.maximum(y, 0.0)

    o_ref[0] = jnp.dot(x_blk[...], w_ref[0], preferred_element_type=jnp.float32)


@jax.jit
def _comb_transform(a0, a1, sl, b, g, be, rm, rv, wall):
    row = pl.BlockSpec((BN_BLK, D), lambda i, r: (i, 0))
    par = pl.BlockSpec((1, D), lambda i, r: (0, 0))
    return pl.pallas_call(
        _comb_transform_body,
        grid=(NB, RT),
        in_specs=[row, row, row, par, par, par, par, par,
                  pl.BlockSpec((1, D, D), lambda i, r: (r, 0, 0))],
        out_specs=pl.BlockSpec((1, BN_BLK, D), lambda i, r: (r, i, 0)),
        out_shape=jax.ShapeDtypeStruct((RT, NP, D), jnp.float32),
        scratch_shapes=[pltpu.VMEM((BN_BLK, D), jnp.float32)],
    )(a0, a1, sl, b.reshape(1, D), g.reshape(1, D), be.reshape(1, D),
      rm.reshape(1, D), rv.reshape(1, D), wall)


def _edge_body(hw, srch, dst4h, eth, out, src_v, et_v, gidx_v, didx_v,
               rows_a, rows_b, rows_c, sem_a, sem_b, sem_c, agg_sh):
    c = lax.axis_index("c")
    s = lax.axis_index("s")
    wid = s * NC + c

    zvec = jnp.zeros((16,), jnp.float32)

    @pl.loop(0, ZR)
    def _(i):
        @pl.loop(0, D, step=16)
        def _(j):
            rows_a[i, pl.ds(j, 16)] = zvec

    base_row = s * RPT

    @pl.loop(0, RPT, step=ZR)
    def _(r):
        pltpu.sync_copy(rows_a.at[pl.ds(0, ZR)],
                        agg_sh.at[pl.ds(base_row + r, ZR)])

    plsc.subcore_barrier()

    ebase = wid * EPW

    @pl.loop(0, NSUP)
    def _(sc):
        pltpu.sync_copy(srch.at[pl.ds(ebase + sc * SCH, SCH)], src_v)
        pltpu.sync_copy(eth.at[pl.ds(ebase + sc * SCH, SCH)], et_v)
        pltpu.sync_copy(dst4h.at[wid, sc], didx_v)

        @pl.loop(0, SCH, step=16)
        def _(j):
            gidx_v[pl.ds(j, 16)] = et_v[pl.ds(j, 16)] * NP + src_v[pl.ds(j, 16)]

        def gather(i, buf, sem):
            pltpu.async_copy(hw.at[gidx_v.at[pl.ds(i * CH, CH)]], buf, sem)

        def wait(buf, sem):
            pltpu.make_async_copy(hw.at[pl.ds(0, CH)], buf, sem).wait()

        def scatter(i, buf):
            pltpu.sync_copy(buf, agg_sh.at[didx_v.at[i]], add=True)

        gather(0, rows_a, sem_a)
        gather(1, rows_b, sem_b)

        @pl.loop(0, NCH_S - 6, step=3)
        def _(i):
            gather(i + 2, rows_c, sem_c)
            wait(rows_a, sem_a)
            scatter(i, rows_a)
            gather(i + 3, rows_a, sem_a)
            wait(rows_b, sem_b)
            scatter(i + 1, rows_b)
            gather(i + 4, rows_b, sem_b)
            wait(rows_c, sem_c)
            scatter(i + 2, rows_c)

        gather(NCH_S - 2, rows_c, sem_c)
        wait(rows_a, sem_a)
        scatter(NCH_S - 4, rows_a)
        gather(NCH_S - 1, rows_a, sem_a)
        wait(rows_b, sem_b)
        scatter(NCH_S - 3, rows_b)
        wait(rows_c, sem_c)
        scatter(NCH_S - 2, rows_c)
        wait(rows_a, sem_a)
        scatter(NCH_S - 1, rows_a)

    plsc.subcore_barrier()

    pltpu.sync_copy(agg_sh.at[pl.ds(base_row, RPT)],
                    out.at[c, pl.ds(base_row, RPT)])


@jax.jit
def _edge_pass(hw_flat, src, dst4, etypes):
    mesh = plsc.VectorSubcoreMesh(core_axis_name="c", subcore_axis_name="s",
                                  num_cores=NC, num_subcores=NS)
    f = pl.kernel(
        _edge_body,
        out_type=jax.ShapeDtypeStruct((NC, NP, D), jnp.float32),
        mesh=mesh,
        scratch_types=[
            pltpu.VMEM((SCH,), jnp.int32),
            pltpu.VMEM((SCH,), jnp.int32),
            pltpu.VMEM((SCH,), jnp.int32),
            pltpu.VMEM((NCH_S, CH), jnp.int32),
            pltpu.VMEM((CH, D), jnp.float32),
            pltpu.VMEM((CH, D), jnp.float32),
            pltpu.VMEM((CH, D), jnp.float32),
            pltpu.SemaphoreType.DMA,
            pltpu.SemaphoreType.DMA,
            pltpu.SemaphoreType.DMA,
            pltpu.VMEM_SHARED((NP, D), jnp.float32),
        ],
    )
    return f(hw_flat, src, dst4, etypes)


def kernel(h, edge_index, etypes, W1, Ws1, b1, gamma1, beta1, rm1, rv1,
           W2, Ws2, b2, gamma2, beta2, rm2, rv2):
    src = edge_index[0]
    dst4 = edge_index[1].reshape(NW, NSUP, NCH_S, CH)

    wall1 = jnp.concatenate([W1, Ws1[None]], axis=0)
    wall2 = jnp.concatenate([W2, Ws2[None]], axis=0)

    hp = jnp.pad(h, ((0, NP - N), (0, 0)))

    hw1 = _transform(hp, wall1)
    agg1 = _edge_pass(hw1.reshape(RT * NP, D), src, dst4, etypes)
    hw2 = _comb_transform(agg1[0], agg1[1], hw1[R], b1, gamma1, beta1,
                          rm1, rv1, wall2)
    agg2 = _edge_pass(hw2.reshape(RT * NP, D), src, dst4, etypes)
    out = _combine(agg2[0], agg2[1], hw2[R], b2, gamma2, beta2, rm2, rv2)
    return out[:N]


# speedup vs baseline: 38.8556x; 1.2364x over previous
import jax
import jax.numpy as jnp
from jax import lax
from jax.experimental import pallas as pl
from jax.experimental.pallas import tpu as pltpu
from jax.experimental.pallas import tpu_sc as plsc

N = 10000
NP = 10240
E = 320000
D = 128
R = 5
RT = R + 1

NC = 2
NS = 16
NW = NC * NS
EPW = E // NW
CH = 80
SCH = 2000
NCH_S = SCH // CH
NSUP = EPW // SCH
RPT = NP // NS
ZR = 80

BN_BLK = 1024
NB = NP // BN_BLK


def _mm_store(res, o_ref):
    for r in range(RT):
        o_ref[r] = res[:, r * D:(r + 1) * D]


def _transform_body(x_ref, w_ref, o_ref):
    xb = x_ref[...].astype(jnp.bfloat16)
    wb = w_ref[...].astype(jnp.bfloat16)
    _mm_store(jnp.dot(xb, wb, preferred_element_type=jnp.float32), o_ref)


@jax.jit
def _transform(x, wcat):
    return pl.pallas_call(
        _transform_body,
        grid=(NB,),
        in_specs=[
            pl.BlockSpec((BN_BLK, D), lambda i: (i, 0)),
            pl.BlockSpec((D, RT * D), lambda i: (0, 0)),
        ],
        out_specs=pl.BlockSpec((RT, BN_BLK, D), lambda i: (0, i, 0)),
        out_shape=jax.ShapeDtypeStruct((RT, NP, D), jnp.float32),
    )(x, wcat)


def _merge(a, a1, sl, b, g, be, rm, rv):
    z = a[0] + a1[0] + sl[...] + b[...]
    z = jnp.where(z > 0, z, 0.01 * z)
    inv = lax.rsqrt(rv[...] + 1e-5)
    y = (z - rm[...]) * inv * g[...] + be[...]
    return jnp.maximum(y, 0.0)


def _combine_body(a, a1, sl, b, g, be, rm, rv, o):
    o[...] = _merge(a, a1, sl, b, g, be, rm, rv)


@jax.jit
def _combine(agg, hwflat, b, g, be, rm, rv):
    par = pl.BlockSpec((1, D), lambda i: (0, 0))
    return pl.pallas_call(
        _combine_body,
        grid=(NB,),
        in_specs=[pl.BlockSpec((1, BN_BLK, D), lambda i: (0, i, 0)),
                  pl.BlockSpec((1, BN_BLK, D), lambda i: (1, i, 0)),
                  pl.BlockSpec((BN_BLK, D), lambda i: (R * NB + i, 0)),
                  par, par, par, par, par],
        out_specs=pl.BlockSpec((BN_BLK, D), lambda i: (i, 0)),
        out_shape=jax.ShapeDtypeStruct((NP, D), jnp.float32),
    )(agg, agg, hwflat, b.reshape(1, D), g.reshape(1, D), be.reshape(1, D),
      rm.reshape(1, D), rv.reshape(1, D))


def _comb_transform_body(a, a1, sl, b, g, be, rm, rv, w_ref, o_ref):
    x = _merge(a, a1, sl, b, g, be, rm, rv)
    wb = w_ref[...].astype(jnp.bfloat16)
    res = jnp.dot(x.astype(jnp.bfloat16), wb,
                  preferred_element_type=jnp.float32)
    _mm_store(res, o_ref)


@jax.jit
def _comb_transform(agg, hwflat, b, g, be, rm, rv, wcat):
    par = pl.BlockSpec((1, D), lambda i: (0, 0))
    return pl.pallas_call(
        _comb_transform_body,
        grid=(NB,),
        in_specs=[pl.BlockSpec((1, BN_BLK, D), lambda i: (0, i, 0)),
                  pl.BlockSpec((1, BN_BLK, D), lambda i: (1, i, 0)),
                  pl.BlockSpec((BN_BLK, D), lambda i: (R * NB + i, 0)),
                  par, par, par, par, par,
                  pl.BlockSpec((D, RT * D), lambda i: (0, 0))],
        out_specs=pl.BlockSpec((RT, BN_BLK, D), lambda i: (0, i, 0)),
        out_shape=jax.ShapeDtypeStruct((RT, NP, D), jnp.float32),
    )(agg, agg, hwflat, b.reshape(1, D), g.reshape(1, D), be.reshape(1, D),
      rm.reshape(1, D), rv.reshape(1, D), wcat)


def _edge_body(hw, srch, dst4h, eth, out, src_v, et_v, gidx_v, didx_v,
               rows_a, rows_b, rows_c, sem_a, sem_b, sem_c, agg_sh):
    c = lax.axis_index("c")
    s = lax.axis_index("s")
    wid = s * NC + c

    zvec = jnp.zeros((16,), jnp.float32)

    @pl.loop(0, ZR)
    def _(i):
        @pl.loop(0, D, step=16)
        def _(j):
            rows_a[i, pl.ds(j, 16)] = zvec

    base_row = s * RPT

    @pl.loop(0, RPT, step=ZR)
    def _(r):
        pltpu.sync_copy(rows_a.at[pl.ds(0, ZR)],
                        agg_sh.at[pl.ds(base_row + r, ZR)])

    plsc.subcore_barrier()

    ebase = wid * EPW

    @pl.loop(0, NSUP)
    def _(sc):
        pltpu.sync_copy(srch.at[pl.ds(ebase + sc * SCH, SCH)], src_v)
        pltpu.sync_copy(eth.at[pl.ds(ebase + sc * SCH, SCH)], et_v)
        pltpu.sync_copy(dst4h.at[wid, sc], didx_v)

        @pl.loop(0, SCH, step=16)
        def _(j):
            gidx_v[pl.ds(j, 16)] = et_v[pl.ds(j, 16)] * NP + src_v[pl.ds(j, 16)]

        def gather(i, buf, sem):
            pltpu.async_copy(hw.at[gidx_v.at[pl.ds(i * CH, CH)]], buf, sem)

        def wait(buf, sem):
            pltpu.make_async_copy(hw.at[pl.ds(0, CH)], buf, sem).wait()

        def scatter(i, buf):
            pltpu.sync_copy(buf, agg_sh.at[didx_v.at[i]], add=True)

        gather(0, rows_a, sem_a)
        gather(1, rows_b, sem_b)

        @pl.loop(0, NCH_S - 6, step=3)
        def _(i):
            gather(i + 2, rows_c, sem_c)
            wait(rows_a, sem_a)
            scatter(i, rows_a)
            gather(i + 3, rows_a, sem_a)
            wait(rows_b, sem_b)
            scatter(i + 1, rows_b)
            gather(i + 4, rows_b, sem_b)
            wait(rows_c, sem_c)
            scatter(i + 2, rows_c)

        gather(NCH_S - 2, rows_c, sem_c)
        wait(rows_a, sem_a)
        scatter(NCH_S - 4, rows_a)
        gather(NCH_S - 1, rows_a, sem_a)
        wait(rows_b, sem_b)
        scatter(NCH_S - 3, rows_b)
        wait(rows_c, sem_c)
        scatter(NCH_S - 2, rows_c)
        wait(rows_a, sem_a)
        scatter(NCH_S - 1, rows_a)

    plsc.subcore_barrier()

    pltpu.sync_copy(agg_sh.at[pl.ds(base_row, RPT)],
                    out.at[c, pl.ds(base_row, RPT)])


@jax.jit
def _edge_pass(hw_flat, src, dst4, etypes):
    mesh = plsc.VectorSubcoreMesh(core_axis_name="c", subcore_axis_name="s",
                                  num_cores=NC, num_subcores=NS)
    f = pl.kernel(
        _edge_body,
        out_type=jax.ShapeDtypeStruct((NC, NP, D), jnp.float32),
        mesh=mesh,
        scratch_types=[
            pltpu.VMEM((SCH,), jnp.int32),
            pltpu.VMEM((SCH,), jnp.int32),
            pltpu.VMEM((SCH,), jnp.int32),
            pltpu.VMEM((NCH_S, CH), jnp.int32),
            pltpu.VMEM((CH, D), jnp.float32),
            pltpu.VMEM((CH, D), jnp.float32),
            pltpu.VMEM((CH, D), jnp.float32),
            pltpu.SemaphoreType.DMA,
            pltpu.SemaphoreType.DMA,
            pltpu.SemaphoreType.DMA,
            pltpu.VMEM_SHARED((NP, D), jnp.float32),
        ],
    )
    return f(hw_flat, src, dst4, etypes)


def kernel(h, edge_index, etypes, W1, Ws1, b1, gamma1, beta1, rm1, rv1,
           W2, Ws2, b2, gamma2, beta2, rm2, rv2):
    src = edge_index[0]
    dst4 = edge_index[1].reshape(NW, NSUP, NCH_S, CH)

    wcat1 = jnp.concatenate([W1, Ws1[None]], axis=0).transpose(1, 0, 2)
    wcat1 = wcat1.reshape(D, RT * D)
    wcat2 = jnp.concatenate([W2, Ws2[None]], axis=0).transpose(1, 0, 2)
    wcat2 = wcat2.reshape(D, RT * D)

    hp = jnp.pad(h, ((0, NP - N), (0, 0)))

    hw1 = _transform(hp, wcat1).reshape(RT * NP, D)
    agg1 = _edge_pass(hw1, src, dst4, etypes)
    hw2 = _comb_transform(agg1, hw1, b1, gamma1, beta1, rm1, rv1, wcat2)
    hw2 = hw2.reshape(RT * NP, D)
    agg2 = _edge_pass(hw2, src, dst4, etypes)
    out = _combine(agg2, hw2, b2, gamma2, beta2, rm2, rv2)
    return out[:N]
